# Initial kernel scaffold; baseline (speedup 1.0000x reference)
import jax
import jax.numpy as jnp
from jax.experimental import pallas as pl


def kernel(x, adj, batch, conv_W0, conv_b0, conv_W1, conv_b1, linnews_W, linnews_b, lin2_W, lin2_b, lin3_W, lin3_b):
    raise NotImplementedError("write your pallas kernel here")


# speedup vs baseline: 3.8565x; 3.8565x over previous
import functools

import jax
import jax.numpy as jnp
from jax.experimental import pallas as pl
from jax.experimental.pallas import tpu as pltpu


N_NODES = 10000
D = 128
G_GRAPHS = 128
ROW_BLK = 2000


def _mm_body(x_ref, w_ref, dinv_ref, g_ref):
    h = jnp.dot(x_ref[...], w_ref[...], preferred_element_type=jnp.float32)
    g_ref[...] = h * dinv_ref[...]


def _matmul_scale(x, w, dinv2d):
    n = x.shape[0]
    grid = (n // ROW_BLK,)
    return pl.pallas_call(
        _mm_body,
        grid=grid,
        in_specs=[
            pl.BlockSpec((ROW_BLK, D), lambda i: (i, 0)),
            pl.BlockSpec((D, D), lambda i: (0, 0)),
            pl.BlockSpec((ROW_BLK, 1), lambda i: (i, 0)),
        ],
        out_specs=pl.BlockSpec((ROW_BLK, D), lambda i: (i, 0)),
        out_shape=jax.ShapeDtypeStruct((n, D), jnp.float32),
    )(x, w, dinv2d)


def _heads_body(hp_ref, nx_ref, w2_ref, b2_ref, wn_ref, bn_ref,
                w3a_ref, w3b_ref, b3_ref, out_ref):
    a = jnp.maximum(
        jnp.dot(hp_ref[...], w2_ref[...], preferred_element_type=jnp.float32)
        + b2_ref[...], 0.0)
    b = jnp.maximum(
        jnp.dot(nx_ref[...], wn_ref[...], preferred_element_type=jnp.float32)
        + bn_ref[...], 0.0)
    z = (jnp.dot(a, w3a_ref[...], preferred_element_type=jnp.float32)
         + jnp.dot(b, w3b_ref[...], preferred_element_type=jnp.float32)
         + b3_ref[...])
    out_ref[...] = jax.nn.sigmoid(z)


def _heads(hp, news_x, lin2_W, lin2_b, linnews_W, linnews_b, lin3_W, lin3_b):
    full = lambda s: pl.BlockSpec(s, lambda: (0,) * len(s))
    return pl.pallas_call(
        _heads_body,
        in_specs=[full((G_GRAPHS, D)), full((G_GRAPHS, D)),
                  full((D, D)), full((1, D)),
                  full((D, D)), full((1, D)),
                  full((D, 1)), full((D, 1)), full((1, 1))],
        out_specs=full((G_GRAPHS, 1)),
        out_shape=jax.ShapeDtypeStruct((G_GRAPHS, 1), jnp.float32),
    )(hp, news_x, lin2_W, lin2_b.reshape(1, D), linnews_W,
      linnews_b.reshape(1, D), lin3_W[:D], lin3_W[D:], lin3_b.reshape(1, 1))


def kernel(x, adj, batch, conv_W0, conv_b0, conv_W1, conv_b1,
           linnews_W, linnews_b, lin2_W, lin2_b, lin3_W, lin3_b):
    src, dst = adj[0], adj[1]
    n = x.shape[0]

    deg = jnp.ones((n,), jnp.float32).at[dst].add(1.0)
    dinv = jax.lax.rsqrt(deg)

    g = _matmul_scale(x, conv_W1, dinv[:, None])

    acc = jnp.zeros((n, D), jnp.float32).at[dst].add(g[src])
    out_conv = jnp.maximum(dinv[:, None] * (acc + g) + conv_b1, 0.0)

    hp = jax.ops.segment_max(out_conv, batch, num_segments=G_GRAPHS)

    diff = batch[1:] - batch[:-1]
    root = jnp.nonzero(diff, size=G_GRAPHS - 1)[0]
    root = jnp.concatenate([jnp.zeros((1,), root.dtype), root + 1], axis=0)
    news_x = x[root]

    return _heads(hp, news_x, lin2_W, lin2_b, linnews_W, linnews_b,
                  lin3_W, lin3_b)


# speedup vs baseline: 8.6879x; 2.2528x over previous
import functools

import jax
import jax.numpy as jnp
from jax import lax
from jax.experimental import pallas as pl
from jax.experimental.pallas import tpu as pltpu
from jax.experimental.pallas import tpu_sc as plsc


N_NODES = 10000
D = 128
G_GRAPHS = 128
ROW_BLK = 2000

NC = 2
NS = 16
NW = NC * NS
CHUNK = 64
CHUNKS_PER_W = 160
GROUP = 32
NGROUPS = CHUNKS_PER_W // GROUP
E_PAD = NW * CHUNKS_PER_W * CHUNK
N_ACC = 10112
ROWS_PER_TILE = N_ACC // NS


def _edge_agg_body(g_hbm, src_hbm, dst_hbm, out_hbm,
                   src_idx, dst_idx, buf0, buf1, acc, sem0, sem1):
    c = lax.axis_index("c")
    s = lax.axis_index("s")
    w = s * NC + c

    def zero_row(i, carry):
        for f in range(D // 16):
            buf0[i, pl.ds(f * 16, 16)] = jnp.zeros((16,), jnp.float32)
        return carry
    lax.fori_loop(0, CHUNK, zero_row, None)
    base = s * ROWS_PER_TILE
    for k in range(ROWS_PER_TILE // CHUNK):
        pltpu.sync_copy(buf0, acc.at[pl.ds(base + k * CHUNK, CHUNK)])
    rem = ROWS_PER_TILE % CHUNK
    if rem:
        pltpu.sync_copy(buf0.at[pl.ds(0, rem)],
                        acc.at[pl.ds(base + ROWS_PER_TILE - rem, rem)])
    plsc.subcore_barrier()

    bufs = (buf0, buf1)
    sems = (sem0, sem1)

    def start(j, b):
        pltpu.async_copy(g_hbm.at[src_idx.at[j]], bufs[b], sems[b])

    def wait(j, b):
        pltpu.make_async_copy(g_hbm.at[src_idx.at[j]], bufs[b], sems[b]).wait()

    for grp in range(NGROUPS):
        g0 = w * CHUNKS_PER_W + grp * GROUP
        pltpu.sync_copy(src_hbm.at[pl.ds(g0, GROUP)], src_idx)
        pltpu.sync_copy(dst_hbm.at[pl.ds(g0, GROUP)], dst_idx)
        start(0, 0)
        start(1, 1)

        def body(i, _):
            for b in range(2):
                j = i * 2 + b
                wait(j, b)
                pltpu.sync_copy(bufs[b], acc.at[dst_idx.at[j]], add=True)
                pl.when(j + 2 < GROUP)(lambda: start(j + 2, b))
            return _
        lax.fori_loop(0, GROUP // 2, body, None)

    plsc.subcore_barrier()

    for k in range(ROWS_PER_TILE // CHUNK):
        b = base + k * CHUNK
        pltpu.sync_copy(acc.at[pl.ds(b, CHUNK)], buf0)
        pltpu.sync_copy(buf0, out_hbm.at[c, pl.ds(b, CHUNK)])
    if rem:
        b = base + ROWS_PER_TILE - rem
        pltpu.sync_copy(acc.at[pl.ds(b, rem)], buf0.at[pl.ds(0, rem)])
        pltpu.sync_copy(buf0.at[pl.ds(0, rem)], out_hbm.at[c, pl.ds(b, rem)])


@functools.partial(
    pl.kernel,
    mesh=plsc.VectorSubcoreMesh(core_axis_name="c", subcore_axis_name="s"),
    out_type=jax.ShapeDtypeStruct((NC, N_ACC, D), jnp.float32),
    scratch_types=[
        pltpu.VMEM((GROUP, CHUNK), jnp.int32),
        pltpu.VMEM((GROUP, CHUNK), jnp.int32),
        pltpu.VMEM((CHUNK, D), jnp.float32),
        pltpu.VMEM((CHUNK, D), jnp.float32),
        pltpu.VMEM_SHARED((N_ACC, D), jnp.float32),
        pltpu.SemaphoreType.DMA,
        pltpu.SemaphoreType.DMA,
    ],
)
def _edge_agg(g_hbm, src_hbm, dst_hbm, out_hbm,
              src_idx, dst_idx, buf0, buf1, acc, sem0, sem1):
    _edge_agg_body(g_hbm, src_hbm, dst_hbm, out_hbm,
                   src_idx, dst_idx, buf0, buf1, acc, sem0, sem1)


def _mm_body(x_ref, w_ref, dinv_ref, g_ref):
    h = jnp.dot(x_ref[...], w_ref[...], preferred_element_type=jnp.float32)
    g_ref[...] = h * dinv_ref[...]


def _matmul_scale(x, w, dinv2d):
    n = x.shape[0]
    grid = (n // ROW_BLK,)
    return pl.pallas_call(
        _mm_body,
        grid=grid,
        in_specs=[
            pl.BlockSpec((ROW_BLK, D), lambda i: (i, 0)),
            pl.BlockSpec((D, D), lambda i: (0, 0)),
            pl.BlockSpec((ROW_BLK, 1), lambda i: (i, 0)),
        ],
        out_specs=pl.BlockSpec((ROW_BLK, D), lambda i: (i, 0)),
        out_shape=jax.ShapeDtypeStruct((n, D), jnp.float32),
    )(x, w, dinv2d)


def _heads_body(hp_ref, nx_ref, w2_ref, b2_ref, wn_ref, bn_ref,
                w3a_ref, w3b_ref, b3_ref, out_ref):
    a = jnp.maximum(
        jnp.dot(hp_ref[...], w2_ref[...], preferred_element_type=jnp.float32)
        + b2_ref[...], 0.0)
    b = jnp.maximum(
        jnp.dot(nx_ref[...], wn_ref[...], preferred_element_type=jnp.float32)
        + bn_ref[...], 0.0)
    z = (jnp.dot(a, w3a_ref[...], preferred_element_type=jnp.float32)
         + jnp.dot(b, w3b_ref[...], preferred_element_type=jnp.float32)
         + b3_ref[...])
    out_ref[...] = jax.nn.sigmoid(z)


def _heads(hp, news_x, lin2_W, lin2_b, linnews_W, linnews_b, lin3_W, lin3_b):
    full = lambda s: pl.BlockSpec(s, lambda: (0,) * len(s))
    return pl.pallas_call(
        _heads_body,
        in_specs=[full((G_GRAPHS, D)), full((G_GRAPHS, D)),
                  full((D, D)), full((1, D)),
                  full((D, D)), full((1, D)),
                  full((D, 1)), full((D, 1)), full((1, 1))],
        out_specs=full((G_GRAPHS, 1)),
        out_shape=jax.ShapeDtypeStruct((G_GRAPHS, 1), jnp.float32),
    )(hp, news_x, lin2_W, lin2_b.reshape(1, D), linnews_W,
      linnews_b.reshape(1, D), lin3_W[:D], lin3_W[D:], lin3_b.reshape(1, 1))


def kernel(x, adj, batch, conv_W0, conv_b0, conv_W1, conv_b1,
           linnews_W, linnews_b, lin2_W, lin2_b, lin3_W, lin3_b):
    src, dst = adj[0], adj[1]
    n = x.shape[0]
    e = src.shape[0]

    deg = jnp.ones((n,), jnp.float32).at[dst].add(1.0)
    dinv = jax.lax.rsqrt(deg)

    g = _matmul_scale(x, conv_W1, dinv[:, None])

    pad = E_PAD - e
    src_p = jnp.concatenate([src, jnp.zeros((pad,), jnp.int32)]).reshape(-1, CHUNK)
    dst_p = jnp.concatenate([dst, jnp.full((pad,), N_NODES, jnp.int32)]).reshape(-1, CHUNK)

    partials = _edge_agg(g, src_p, dst_p)
    acc = partials[0, :n] + partials[1, :n]

    out_conv = jnp.maximum(dinv[:, None] * (acc + g) + conv_b1, 0.0)

    hp = jax.ops.segment_max(out_conv, batch, num_segments=G_GRAPHS)

    diff = batch[1:] - batch[:-1]
    root = jnp.nonzero(diff, size=G_GRAPHS - 1)[0]
    root = jnp.concatenate([jnp.zeros((1,), root.dtype), root + 1], axis=0)
    news_x = x[root]

    return _heads(hp, news_x, lin2_W, lin2_b, linnews_W, linnews_b,
                  lin3_W, lin3_b)


# speedup vs baseline: 8.7389x; 1.0059x over previous
import functools

import jax
import jax.numpy as jnp
from jax import lax
from jax.experimental import pallas as pl
from jax.experimental.pallas import tpu as pltpu
from jax.experimental.pallas import tpu_sc as plsc


N_NODES = 10000
D = 128
G_GRAPHS = 128
ROW_BLK = 2000

NC = 2
NS = 16
NW = NC * NS
CHUNK = 64
CHUNKS_PER_W = 160
GROUP = 32
NGROUPS = CHUNKS_PER_W // GROUP
E_PAD = NW * CHUNKS_PER_W * CHUNK
N_ACC = 10112
ROWS_PER_TILE = N_ACC // NS


def _edge_agg_body(g_hbm, src_hbm, dst_hbm, out_hbm,
                   src_idx, dst_idx, buf0, buf1, acc, sem0, sem1):
    c = lax.axis_index("c")
    s = lax.axis_index("s")
    w = s * NC + c

    with jax.named_scope("agg_zero"):
        def zero_row(i, carry):
            for f in range(D // 16):
                buf0[i, pl.ds(f * 16, 16)] = jnp.zeros((16,), jnp.float32)
            return carry
        lax.fori_loop(0, CHUNK, zero_row, None)
        base = s * ROWS_PER_TILE
        for k in range(ROWS_PER_TILE // CHUNK):
            pltpu.sync_copy(buf0, acc.at[pl.ds(base + k * CHUNK, CHUNK)])
        rem = ROWS_PER_TILE % CHUNK
        if rem:
            pltpu.sync_copy(buf0.at[pl.ds(0, rem)],
                            acc.at[pl.ds(base + ROWS_PER_TILE - rem, rem)])
        plsc.subcore_barrier()

    bufs = (buf0, buf1)
    sems = (sem0, sem1)

    def start(j, b):
        pltpu.async_copy(g_hbm.at[src_idx.at[j]], bufs[b], sems[b])

    def wait(j, b):
        pltpu.make_async_copy(g_hbm.at[src_idx.at[j]], bufs[b], sems[b]).wait()

    with jax.named_scope("agg_edges"):
        for grp in range(NGROUPS):
            g0 = w * CHUNKS_PER_W + grp * GROUP
            pltpu.sync_copy(src_hbm.at[pl.ds(g0, GROUP)], src_idx)
            pltpu.sync_copy(dst_hbm.at[pl.ds(g0, GROUP)], dst_idx)
            start(0, 0)
            start(1, 1)

            def body(i, _):
                for b in range(2):
                    j = i * 2 + b
                    wait(j, b)
                    pltpu.sync_copy(bufs[b], acc.at[dst_idx.at[j]], add=True)
                    pl.when(j + 2 < GROUP)(lambda: start(j + 2, b))
                return _
            lax.fori_loop(0, GROUP // 2, body, None)

        plsc.subcore_barrier()

    with jax.named_scope("agg_writeout"):
        for k in range(ROWS_PER_TILE // CHUNK):
            b = base + k * CHUNK
            pltpu.sync_copy(acc.at[pl.ds(b, CHUNK)], buf0)
            pltpu.sync_copy(buf0, out_hbm.at[c, pl.ds(b, CHUNK)])
        if rem:
            b = base + ROWS_PER_TILE - rem
            pltpu.sync_copy(acc.at[pl.ds(b, rem)], buf0.at[pl.ds(0, rem)])
            pltpu.sync_copy(buf0.at[pl.ds(0, rem)],
                            out_hbm.at[c, pl.ds(b, rem)])


@functools.partial(
    pl.kernel,
    mesh=plsc.VectorSubcoreMesh(core_axis_name="c", subcore_axis_name="s"),
    out_type=jax.ShapeDtypeStruct((NC, N_ACC, D), jnp.float32),
    scratch_types=[
        pltpu.VMEM((GROUP, CHUNK), jnp.int32),
        pltpu.VMEM((GROUP, CHUNK), jnp.int32),
        pltpu.VMEM((CHUNK, D), jnp.float32),
        pltpu.VMEM((CHUNK, D), jnp.float32),
        pltpu.VMEM_SHARED((N_ACC, D), jnp.float32),
        pltpu.SemaphoreType.DMA,
        pltpu.SemaphoreType.DMA,
    ],
)
def _edge_agg(g_hbm, src_hbm, dst_hbm, out_hbm,
              src_idx, dst_idx, buf0, buf1, acc, sem0, sem1):
    _edge_agg_body(g_hbm, src_hbm, dst_hbm, out_hbm,
                   src_idx, dst_idx, buf0, buf1, acc, sem0, sem1)


def _mm_body(x_ref, w_ref, dinv_ref, g_ref):
    h = jnp.dot(x_ref[...], w_ref[...], preferred_element_type=jnp.float32)
    g_ref[...] = h * dinv_ref[...]


def _matmul_scale(x, w, dinv2d):
    n = x.shape[0]
    grid = (n // ROW_BLK,)
    return pl.pallas_call(
        _mm_body,
        grid=grid,
        in_specs=[
            pl.BlockSpec((ROW_BLK, D), lambda i: (i, 0)),
            pl.BlockSpec((D, D), lambda i: (0, 0)),
            pl.BlockSpec((ROW_BLK, 1), lambda i: (i, 0)),
        ],
        out_specs=pl.BlockSpec((ROW_BLK, D), lambda i: (i, 0)),
        out_shape=jax.ShapeDtypeStruct((n, D), jnp.float32),
    )(x, w, dinv2d)


def _heads_body(hp_ref, nx_ref, w2_ref, b2_ref, wn_ref, bn_ref,
                w3a_ref, w3b_ref, b3_ref, out_ref):
    a = jnp.maximum(
        jnp.dot(hp_ref[...], w2_ref[...], preferred_element_type=jnp.float32)
        + b2_ref[...], 0.0)
    b = jnp.maximum(
        jnp.dot(nx_ref[...], wn_ref[...], preferred_element_type=jnp.float32)
        + bn_ref[...], 0.0)
    z = (jnp.dot(a, w3a_ref[...], preferred_element_type=jnp.float32)
         + jnp.dot(b, w3b_ref[...], preferred_element_type=jnp.float32)
         + b3_ref[...])
    out_ref[...] = jax.nn.sigmoid(z)


def _heads(hp, news_x, lin2_W, lin2_b, linnews_W, linnews_b, lin3_W, lin3_b):
    full = lambda s: pl.BlockSpec(s, lambda: (0,) * len(s))
    return pl.pallas_call(
        _heads_body,
        in_specs=[full((G_GRAPHS, D)), full((G_GRAPHS, D)),
                  full((D, D)), full((1, D)),
                  full((D, D)), full((1, D)),
                  full((D, 1)), full((D, 1)), full((1, 1))],
        out_specs=full((G_GRAPHS, 1)),
        out_shape=jax.ShapeDtypeStruct((G_GRAPHS, 1), jnp.float32),
    )(hp, news_x, lin2_W, lin2_b.reshape(1, D), linnews_W,
      linnews_b.reshape(1, D), lin3_W[:D], lin3_W[D:], lin3_b.reshape(1, 1))


def kernel(x, adj, batch, conv_W0, conv_b0, conv_W1, conv_b1,
           linnews_W, linnews_b, lin2_W, lin2_b, lin3_W, lin3_b):
    src, dst = adj[0], adj[1]
    n = x.shape[0]
    e = src.shape[0]

    deg = jnp.ones((n,), jnp.float32).at[dst].add(1.0)
    dinv = jax.lax.rsqrt(deg)

    g = _matmul_scale(x, conv_W1, dinv[:, None])

    pad = E_PAD - e
    src_p = jnp.concatenate([src, jnp.zeros((pad,), jnp.int32)]).reshape(-1, CHUNK)
    dst_p = jnp.concatenate([dst, jnp.full((pad,), N_NODES, jnp.int32)]).reshape(-1, CHUNK)

    partials = _edge_agg(g, src_p, dst_p)
    acc = partials[0, :n] + partials[1, :n]

    out_conv = jnp.maximum(dinv[:, None] * (acc + g) + conv_b1, 0.0)

    hp = jax.ops.segment_max(out_conv, batch, num_segments=G_GRAPHS)

    diff = batch[1:] - batch[:-1]
    root = jnp.nonzero(diff, size=G_GRAPHS - 1)[0]
    root = jnp.concatenate([jnp.zeros((1,), root.dtype), root + 1], axis=0)
    news_x = x[root]

    return _heads(hp, news_x, lin2_W, lin2_b, linnews_W, linnews_b,
                  lin3_W, lin3_b)


# speedup vs baseline: 14.0469x; 1.6074x over previous
import functools

import jax
import jax.numpy as jnp
from jax import lax
from jax.experimental import pallas as pl
from jax.experimental.pallas import tpu as pltpu
from jax.experimental.pallas import tpu_sc as plsc


N_NODES = 10000
D = 128
G_GRAPHS = 128
ROW_BLK = 2000

NC = 2
NS = 16
NW = NC * NS
CHUNK = 64
CHUNKS_PER_S = 320
CPW0 = 224
CPW1 = 96
GROUP = 32
E_PAD = NS * CHUNKS_PER_S * CHUNK
EDGES_PER_HIST_W = E_PAD // NW
HR = 80
N_ACC = 10112
ROWS_PER_TILE = N_ACC // NS


def _edge_agg_body(g_hbm, src_hbm, dst_hbm, out_hbm,
                   src_idx, dst_idx, buf0, buf1, acc, sem0, sem1):
    c = lax.axis_index("c")
    s = lax.axis_index("s")
    w = s * NC + c

    with jax.named_scope("agg_zero"):
        def zero_row(i, carry):
            for f in range(D // 16):
                buf0[i, pl.ds(f * 16, 16)] = jnp.zeros((16,), jnp.float32)
            return carry
        lax.fori_loop(0, CHUNK, zero_row, None)
        base = s * ROWS_PER_TILE
        for k in range(ROWS_PER_TILE // CHUNK):
            pltpu.sync_copy(buf0, acc.at[pl.ds(base + k * CHUNK, CHUNK)])
        rem = ROWS_PER_TILE % CHUNK
        if rem:
            pltpu.sync_copy(buf0.at[pl.ds(0, rem)],
                            acc.at[pl.ds(base + ROWS_PER_TILE - rem, rem)])
        plsc.subcore_barrier()

    bufs = (buf0, buf1)
    sems = (sem0, sem1)

    def start(j, b):
        pltpu.async_copy(g_hbm.at[src_idx.at[j]], bufs[b], sems[b])

    def wait(j, b):
        pltpu.make_async_copy(g_hbm.at[src_idx.at[j]], bufs[b], sems[b]).wait()

    def run_groups(first_chunk, ngroups):
        for grp in range(ngroups):
            g0 = first_chunk + grp * GROUP
            pltpu.sync_copy(src_hbm.at[pl.ds(g0, GROUP)], src_idx)
            pltpu.sync_copy(dst_hbm.at[pl.ds(g0, GROUP)], dst_idx)
            start(0, 0)
            start(1, 1)

            def body(i, _):
                for b in range(2):
                    j = i * 2 + b
                    wait(j, b)
                    pltpu.sync_copy(bufs[b], acc.at[dst_idx.at[j]], add=True)
                    pl.when(j + 2 < GROUP)(lambda: start(j + 2, b))
                return _
            lax.fori_loop(0, GROUP // 2, body, None)

    with jax.named_scope("agg_edges"):
        base_chunk = s * CHUNKS_PER_S
        pl.when(c == 0)(lambda: run_groups(base_chunk, CPW0 // GROUP))
        pl.when(c == 1)(lambda: run_groups(base_chunk + CPW0, CPW1 // GROUP))
        plsc.subcore_barrier()

    with jax.named_scope("agg_writeout"):
        for k in range(ROWS_PER_TILE // CHUNK):
            b = base + k * CHUNK
            pltpu.sync_copy(acc.at[pl.ds(b, CHUNK)], buf0)
            pltpu.sync_copy(buf0, out_hbm.at[c, pl.ds(b, CHUNK)])
        if rem:
            b = base + ROWS_PER_TILE - rem
            pltpu.sync_copy(acc.at[pl.ds(b, rem)], buf0.at[pl.ds(0, rem)])
            pltpu.sync_copy(buf0.at[pl.ds(0, rem)],
                            out_hbm.at[c, pl.ds(b, rem)])


@functools.partial(
    pl.kernel,
    mesh=plsc.VectorSubcoreMesh(core_axis_name="c", subcore_axis_name="s"),
    out_type=jax.ShapeDtypeStruct((NC, N_ACC, D), jnp.float32),
    scratch_types=[
        pltpu.VMEM((GROUP, CHUNK), jnp.int32),
        pltpu.VMEM((GROUP, CHUNK), jnp.int32),
        pltpu.VMEM((CHUNK, D), jnp.float32),
        pltpu.VMEM((CHUNK, D), jnp.float32),
        pltpu.VMEM_SHARED((N_ACC, D), jnp.float32),
        pltpu.SemaphoreType.DMA,
        pltpu.SemaphoreType.DMA,
    ],
)
def _edge_agg(g_hbm, src_hbm, dst_hbm, out_hbm,
              src_idx, dst_idx, buf0, buf1, acc, sem0, sem1):
    _edge_agg_body(g_hbm, src_hbm, dst_hbm, out_hbm,
                   src_idx, dst_idx, buf0, buf1, acc, sem0, sem1)


DEG_HIST = NW * (E_PAD // NW // NW)
HIST_N = 10240
DCH = 160
DGROUP = 32


def _deg_body(dst_hbm, degp_hbm, dst_idx, ones_v, zb, wb, hist, sem):
    c = lax.axis_index("c")
    s = lax.axis_index("s")
    w = s * NC + c
    zero16 = jnp.zeros((16,), jnp.float32)

    def zrow(i, carry):
        zb[pl.ds(i * 16, 16)] = zero16
        return carry
    lax.fori_loop(0, (HIST_N // NS) // 16, zrow, None)
    pltpu.sync_copy(zb, hist.at[pl.ds(s * (HIST_N // NS), HIST_N // NS)])
    for k in range(CHUNK // 16):
        ones_v[pl.ds(k * 16, 16)] = jnp.ones((16,), jnp.float32)
    plsc.subcore_barrier()

    for grp in range(DCH // DGROUP):
        g0 = w * DCH + grp * DGROUP
        pltpu.sync_copy(dst_hbm.at[pl.ds(g0, DGROUP)], dst_idx)

        def body(j, carry):
            pltpu.sync_copy(ones_v, hist.at[dst_idx.at[j]], add=True)
            return carry
        lax.fori_loop(0, DGROUP, body, None)
    plsc.subcore_barrier()

    def wout():
        q = HIST_N // 4
        pltpu.sync_copy(hist.at[pl.ds(s * q, q)], wb)
        pltpu.sync_copy(wb, degp_hbm.at[c, pl.ds(s * q, q)])
    pl.when(s < 4)(wout)


@functools.partial(
    pl.kernel,
    mesh=plsc.VectorSubcoreMesh(core_axis_name="c", subcore_axis_name="s"),
    out_type=jax.ShapeDtypeStruct((NC, HIST_N), jnp.float32),
    scratch_types=[
        pltpu.VMEM((DGROUP, CHUNK), jnp.int32),
        pltpu.VMEM((CHUNK,), jnp.float32),
        pltpu.VMEM((HIST_N // NS,), jnp.float32),
        pltpu.VMEM((HIST_N // 4,), jnp.float32),
        pltpu.VMEM_SHARED((HIST_N,), jnp.float32),
        pltpu.SemaphoreType.DMA,
    ],
)
def _deg_hist(dst_hbm, degp_hbm, dst_idx, ones_v, zb, wb, hist, sem):
    _deg_body(dst_hbm, degp_hbm, dst_idx, ones_v, zb, wb, hist, sem)


def _mm_body(x_ref, w_ref, dinv_ref, g_ref):
    h = jnp.dot(x_ref[...], w_ref[...], preferred_element_type=jnp.float32)
    g_ref[...] = h * dinv_ref[...]


def _matmul_scale(x, w, dinv2d):
    n = x.shape[0]
    grid = (n // ROW_BLK,)
    return pl.pallas_call(
        _mm_body,
        grid=grid,
        in_specs=[
            pl.BlockSpec((ROW_BLK, D), lambda i: (i, 0)),
            pl.BlockSpec((D, D), lambda i: (0, 0)),
            pl.BlockSpec((ROW_BLK, 1), lambda i: (i, 0)),
        ],
        out_specs=pl.BlockSpec((ROW_BLK, D), lambda i: (i, 0)),
        out_shape=jax.ShapeDtypeStruct((n, D), jnp.float32),
    )(x, w, dinv2d)


def _heads_body(hp_ref, nx_ref, w2_ref, b2_ref, wn_ref, bn_ref,
                w3a_ref, w3b_ref, b3_ref, out_ref):
    a = jnp.maximum(
        jnp.dot(hp_ref[...], w2_ref[...], preferred_element_type=jnp.float32)
        + b2_ref[...], 0.0)
    b = jnp.maximum(
        jnp.dot(nx_ref[...], wn_ref[...], preferred_element_type=jnp.float32)
        + bn_ref[...], 0.0)
    z = (jnp.dot(a, w3a_ref[...], preferred_element_type=jnp.float32)
         + jnp.dot(b, w3b_ref[...], preferred_element_type=jnp.float32)
         + b3_ref[...])
    out_ref[...] = jax.nn.sigmoid(z)


def _heads(hp, news_x, lin2_W, lin2_b, linnews_W, linnews_b, lin3_W, lin3_b):
    full = lambda s: pl.BlockSpec(s, lambda: (0,) * len(s))
    return pl.pallas_call(
        _heads_body,
        in_specs=[full((G_GRAPHS, D)), full((G_GRAPHS, D)),
                  full((D, D)), full((1, D)),
                  full((D, D)), full((1, D)),
                  full((D, 1)), full((D, 1)), full((1, 1))],
        out_specs=full((G_GRAPHS, 1)),
        out_shape=jax.ShapeDtypeStruct((G_GRAPHS, 1), jnp.float32),
    )(hp, news_x, lin2_W, lin2_b.reshape(1, D), linnews_W,
      linnews_b.reshape(1, D), lin3_W[:D], lin3_W[D:], lin3_b.reshape(1, 1))


def kernel(x, adj, batch, conv_W0, conv_b0, conv_W1, conv_b1,
           linnews_W, linnews_b, lin2_W, lin2_b, lin3_W, lin3_b):
    src, dst = adj[0], adj[1]
    n = x.shape[0]
    e = src.shape[0]

    pad = E_PAD - e
    src_p = jnp.concatenate([src, jnp.zeros((pad,), jnp.int32)]).reshape(-1, CHUNK)
    dst_flat = jnp.concatenate([dst, jnp.full((pad,), N_NODES, jnp.int32)])
    dst_p = dst_flat.reshape(-1, CHUNK)

    degp = _deg_hist(dst_p)
    deg = 1.0 + (degp[0] + degp[1])[:n]
    dinv = jax.lax.rsqrt(deg)

    g = _matmul_scale(x, conv_W1, dinv[:, None])

    partials = _edge_agg(g, src_p, dst_p)
    acc = partials[0, :n] + partials[1, :n]

    out_conv = jnp.maximum(dinv[:, None] * (acc + g) + conv_b1, 0.0)

    hp = jax.ops.segment_max(out_conv, batch, num_segments=G_GRAPHS)

    diff = batch[1:] - batch[:-1]
    root = jnp.nonzero(diff, size=G_GRAPHS - 1)[0]
    root = jnp.concatenate([jnp.zeros((1,), root.dtype), root + 1], axis=0)
    news_x = x[root]

    return _heads(hp, news_x, lin2_W, lin2_b, linnews_W, linnews_b,
                  lin3_W, lin3_b)


# speedup vs baseline: 26.6783x; 1.8992x over previous
import functools

import jax
import jax.numpy as jnp
from jax import lax
from jax.experimental import pallas as pl
from jax.experimental.pallas import tpu as pltpu
from jax.experimental.pallas import tpu_sc as plsc


N_NODES = 10000
D = 128
G_GRAPHS = 128
ROW_BLK = 2000

NC = 2
NS = 16
NW = NC * NS
CHUNK = 64
CHUNKS_PER_S = 320
CPW0 = 160
CPW1 = 160
GROUP = 32
E_PAD = NS * CHUNKS_PER_S * CHUNK
EDGES_PER_HIST_W = E_PAD // NW
HR = 80
N_ACC = 10112
ROWS_PER_TILE = N_ACC // NS


def _edge_agg_body(g_hbm, src_hbm, dst_hbm, out_hbm,
                   src_idx, dst_idx, buf0, buf1, acc, sem0, sem1):
    c = lax.axis_index("c")
    s = lax.axis_index("s")
    w = s * NC + c

    with jax.named_scope("agg_zero"):
        def zero_row(i, carry):
            for f in range(D // 16):
                buf0[i, pl.ds(f * 16, 16)] = jnp.zeros((16,), jnp.float32)
            return carry
        lax.fori_loop(0, CHUNK, zero_row, None)
        base = s * ROWS_PER_TILE
        for k in range(ROWS_PER_TILE // CHUNK):
            pltpu.sync_copy(buf0, acc.at[pl.ds(base + k * CHUNK, CHUNK)])
        rem = ROWS_PER_TILE % CHUNK
        if rem:
            pltpu.sync_copy(buf0.at[pl.ds(0, rem)],
                            acc.at[pl.ds(base + ROWS_PER_TILE - rem, rem)])
        plsc.subcore_barrier()

    bufs = (buf0, buf1)
    sems = (sem0, sem1)

    def start(j, b):
        pltpu.async_copy(g_hbm.at[src_idx.at[j]], bufs[b], sems[b])

    def wait(j, b):
        pltpu.make_async_copy(g_hbm.at[src_idx.at[j]], bufs[b], sems[b]).wait()

    def run_groups(first_chunk, ngroups):
        for grp in range(ngroups):
            g0 = first_chunk + grp * GROUP
            pltpu.sync_copy(src_hbm.at[pl.ds(g0, GROUP)], src_idx)
            pltpu.sync_copy(dst_hbm.at[pl.ds(g0, GROUP)], dst_idx)
            start(0, 0)
            start(1, 1)

            def body(i, _):
                for b in range(2):
                    j = i * 2 + b
                    wait(j, b)
                    pltpu.sync_copy(bufs[b], acc.at[dst_idx.at[j]], add=True)
                    pl.when(j + 2 < GROUP)(lambda: start(j + 2, b))
                return _
            lax.fori_loop(0, GROUP // 2, body, None)

    with jax.named_scope("agg_edges"):
        base_chunk = s * CHUNKS_PER_S
        pl.when(c == 0)(lambda: run_groups(base_chunk, CPW0 // GROUP))
        pl.when(c == 1)(lambda: run_groups(base_chunk + CPW0, CPW1 // GROUP))
        plsc.subcore_barrier()

    with jax.named_scope("agg_writeout"):
        for k in range(ROWS_PER_TILE // CHUNK):
            b = base + k * CHUNK
            pltpu.sync_copy(acc.at[pl.ds(b, CHUNK)], buf0)
            pltpu.sync_copy(buf0, out_hbm.at[c, pl.ds(b, CHUNK)])
        if rem:
            b = base + ROWS_PER_TILE - rem
            pltpu.sync_copy(acc.at[pl.ds(b, rem)], buf0.at[pl.ds(0, rem)])
            pltpu.sync_copy(buf0.at[pl.ds(0, rem)],
                            out_hbm.at[c, pl.ds(b, rem)])


@functools.partial(
    pl.kernel,
    mesh=plsc.VectorSubcoreMesh(core_axis_name="c", subcore_axis_name="s"),
    out_type=jax.ShapeDtypeStruct((NC, N_ACC, D), jnp.float32),
    scratch_types=[
        pltpu.VMEM((GROUP, CHUNK), jnp.int32),
        pltpu.VMEM((GROUP, CHUNK), jnp.int32),
        pltpu.VMEM((CHUNK, D), jnp.float32),
        pltpu.VMEM((CHUNK, D), jnp.float32),
        pltpu.VMEM_SHARED((N_ACC, D), jnp.float32),
        pltpu.SemaphoreType.DMA,
        pltpu.SemaphoreType.DMA,
    ],
)
def _edge_agg(g_hbm, src_hbm, dst_hbm, out_hbm,
              src_idx, dst_idx, buf0, buf1, acc, sem0, sem1):
    _edge_agg_body(g_hbm, src_hbm, dst_hbm, out_hbm,
                   src_idx, dst_idx, buf0, buf1, acc, sem0, sem1)


DEG_HIST = NW * (E_PAD // NW // NW)
HIST_N = 10240
DCH = 160
DGROUP = 32


def _deg_body(dst_hbm, degp_hbm, dst_idx, ones_v, zb, wb, hist, sem):
    c = lax.axis_index("c")
    s = lax.axis_index("s")
    w = s * NC + c
    zero16 = jnp.zeros((16,), jnp.float32)

    def zrow(i, carry):
        zb[pl.ds(i * 16, 16)] = zero16
        return carry
    lax.fori_loop(0, (HIST_N // NS) // 16, zrow, None)
    pltpu.sync_copy(zb, hist.at[pl.ds(s * (HIST_N // NS), HIST_N // NS)])
    for k in range(CHUNK // 16):
        ones_v[pl.ds(k * 16, 16)] = jnp.ones((16,), jnp.float32)
    plsc.subcore_barrier()

    for grp in range(DCH // DGROUP):
        g0 = w * DCH + grp * DGROUP
        pltpu.sync_copy(dst_hbm.at[pl.ds(g0, DGROUP)], dst_idx)

        def body(j, carry):
            pltpu.sync_copy(ones_v, hist.at[dst_idx.at[j]], add=True)
            return carry
        lax.fori_loop(0, DGROUP, body, None)
    plsc.subcore_barrier()

    def wout():
        q = HIST_N // 4
        pltpu.sync_copy(hist.at[pl.ds(s * q, q)], wb)
        pltpu.sync_copy(wb, degp_hbm.at[c, pl.ds(s * q, q)])
    pl.when(s < 4)(wout)


@functools.partial(
    pl.kernel,
    mesh=plsc.VectorSubcoreMesh(core_axis_name="c", subcore_axis_name="s"),
    out_type=jax.ShapeDtypeStruct((NC, HIST_N), jnp.float32),
    scratch_types=[
        pltpu.VMEM((DGROUP, CHUNK), jnp.int32),
        pltpu.VMEM((CHUNK,), jnp.float32),
        pltpu.VMEM((HIST_N // NS,), jnp.float32),
        pltpu.VMEM((HIST_N // 4,), jnp.float32),
        pltpu.VMEM_SHARED((HIST_N,), jnp.float32),
        pltpu.SemaphoreType.DMA,
    ],
)
def _deg_hist(dst_hbm, degp_hbm, dst_idx, ones_v, zb, wb, hist, sem):
    _deg_body(dst_hbm, degp_hbm, dst_idx, ones_v, zb, wb, hist, sem)


def _mm_body(x_ref, w_ref, dinv_ref, g_ref):
    h = jnp.dot(x_ref[...], w_ref[...], preferred_element_type=jnp.float32)
    g_ref[...] = h * dinv_ref[...]


def _matmul_scale(x, w, dinv2d):
    n = x.shape[0]
    grid = (n // ROW_BLK,)
    return pl.pallas_call(
        _mm_body,
        grid=grid,
        in_specs=[
            pl.BlockSpec((ROW_BLK, D), lambda i: (i, 0)),
            pl.BlockSpec((D, D), lambda i: (0, 0)),
            pl.BlockSpec((ROW_BLK, 1), lambda i: (i, 0)),
        ],
        out_specs=pl.BlockSpec((ROW_BLK, D), lambda i: (i, 0)),
        out_shape=jax.ShapeDtypeStruct((n, D), jnp.float32),
    )(x, w, dinv2d)


def _heads_body(hp_ref, nx_ref, w2_ref, b2_ref, wn_ref, bn_ref,
                w3a_ref, w3b_ref, b3_ref, out_ref):
    a = jnp.maximum(
        jnp.dot(hp_ref[...], w2_ref[...], preferred_element_type=jnp.float32)
        + b2_ref[...], 0.0)
    b = jnp.maximum(
        jnp.dot(nx_ref[...], wn_ref[...], preferred_element_type=jnp.float32)
        + bn_ref[...], 0.0)
    z = (jnp.dot(a, w3a_ref[...], preferred_element_type=jnp.float32)
         + jnp.dot(b, w3b_ref[...], preferred_element_type=jnp.float32)
         + b3_ref[...])
    out_ref[...] = jax.nn.sigmoid(z)


def _heads(hp, news_x, lin2_W, lin2_b, linnews_W, linnews_b, lin3_W, lin3_b):
    full = lambda s: pl.BlockSpec(s, lambda: (0,) * len(s))
    return pl.pallas_call(
        _heads_body,
        in_specs=[full((G_GRAPHS, D)), full((G_GRAPHS, D)),
                  full((D, D)), full((1, D)),
                  full((D, D)), full((1, D)),
                  full((D, 1)), full((D, 1)), full((1, 1))],
        out_specs=full((G_GRAPHS, 1)),
        out_shape=jax.ShapeDtypeStruct((G_GRAPHS, 1), jnp.float32),
    )(hp, news_x, lin2_W, lin2_b.reshape(1, D), linnews_W,
      linnews_b.reshape(1, D), lin3_W[:D], lin3_W[D:], lin3_b.reshape(1, 1))


def kernel(x, adj, batch, conv_W0, conv_b0, conv_W1, conv_b1,
           linnews_W, linnews_b, lin2_W, lin2_b, lin3_W, lin3_b):
    src, dst = adj[0], adj[1]
    n = x.shape[0]
    e = src.shape[0]

    pad = E_PAD - e
    pad_ar = jnp.arange(pad, dtype=jnp.int32)
    src_p = jnp.concatenate([src, pad_ar % n]).reshape(-1, CHUNK)
    dst_flat = jnp.concatenate([dst, N_NODES + pad_ar % (N_ACC - N_NODES)])
    dst_p = dst_flat.reshape(-1, CHUNK)

    degp = _deg_hist(dst_p)
    deg = 1.0 + (degp[0] + degp[1])[:n]
    dinv = jax.lax.rsqrt(deg)

    g = _matmul_scale(x, conv_W1, dinv[:, None])

    partials = _edge_agg(g, src_p, dst_p)
    acc = partials[0, :n] + partials[1, :n]

    out_conv = jnp.maximum(dinv[:, None] * (acc + g) + conv_b1, 0.0)

    hp = jax.ops.segment_max(out_conv, batch, num_segments=G_GRAPHS)

    diff = batch[1:] - batch[:-1]
    root = jnp.nonzero(diff, size=G_GRAPHS - 1)[0]
    root = jnp.concatenate([jnp.zeros((1,), root.dtype), root + 1], axis=0)
    news_x = x[root]

    return _heads(hp, news_x, lin2_W, lin2_b, linnews_W, linnews_b,
                  lin3_W, lin3_b)
